# baseline (speedup 1.0000x reference)
import functools

import jax
import jax.numpy as jnp
from jax import lax
from jax.experimental import pallas as pl
from jax.experimental.pallas import tpu as pltpu
from jax.experimental.pallas import tpu_sc as plsc

B, N, CPT = 8, 4096, 64
NPOINT, NSAMPLE = 512, 32
BN_EPS = 1e-3
CIN = 3 + CPT
CPAD = 128
QB = 128
QK = 256
TOT = B * NPOINT * NSAMPLE


def _fps_body(xyzt_ref, idx_ref, nxq_ref):
    x = xyzt_ref[0]
    y = xyzt_ref[1]
    z = xyzt_ref[2]
    lane = lax.broadcasted_iota(jnp.int32, (B, N), 1)
    slot = lax.broadcasted_iota(jnp.int32, (B, NPOINT), 1)

    cx0 = x[:, 0:1]
    cy0 = y[:, 0:1]
    cz0 = z[:, 0:1]
    idxs0 = jnp.zeros((B, NPOINT), jnp.int32)
    qx0 = jnp.where(slot == 0, cx0, 0.0)
    qy0 = jnp.where(slot == 0, cy0, 0.0)
    qz0 = jnp.where(slot == 0, cz0, 0.0)
    dists0 = jnp.full((B, N), 1e10, jnp.float32)

    def body(i, st):
        dists, idxs, qx, qy, qz, cx, cy, cz = st
        dx = x - cx
        dy = y - cy
        dz = z - cz
        d = dx * dx + dy * dy + dz * dz
        dists = jnp.minimum(dists, d)
        m = jnp.max(dists, axis=1, keepdims=True)
        nxt = jnp.min(jnp.where(dists == m, lane, N), axis=1, keepdims=True)
        sel = lane == nxt
        cx = jnp.sum(jnp.where(sel, x, 0.0), axis=1, keepdims=True)
        cy = jnp.sum(jnp.where(sel, y, 0.0), axis=1, keepdims=True)
        cz = jnp.sum(jnp.where(sel, z, 0.0), axis=1, keepdims=True)
        put = slot == i
        idxs = jnp.where(put, nxt, idxs)
        qx = jnp.where(put, cx, qx)
        qy = jnp.where(put, cy, qy)
        qz = jnp.where(put, cz, qz)
        return (dists, idxs, qx, qy, qz, cx, cy, cz)

    st = lax.fori_loop(1, NPOINT, body,
                       (dists0, idxs0, qx0, qy0, qz0, cx0, cy0, cz0))
    _, idxs, qx, qy, qz, _, _, _ = st
    idx_ref[...] = idxs
    nxq_ref[0] = qx
    nxq_ref[1] = qy
    nxq_ref[2] = qz


def _fps(xyzt, interpret=False):
    return pl.pallas_call(
        _fps_body,
        out_shape=(jax.ShapeDtypeStruct((B, NPOINT), jnp.int32),
                   jax.ShapeDtypeStruct((3, B, NPOINT), jnp.float32)),
        interpret=interpret,
    )(xyzt)


def _knn_body(xyz_ref, nxq_ref, idx_ref, gidx_ref):
    b = pl.program_id(0)
    ones_c = jnp.ones((N, 1), jnp.float32)
    q = nxq_ref[0]
    ones_r = jnp.ones((1, QK), jnp.float32)
    dims = (((1,), (0,)), ((), ()))

    def diff(c):
        lhs = jnp.concatenate([xyz_ref[0, :, c:c + 1], ones_c], axis=1)
        rhs = jnp.concatenate([ones_r, -q[c:c + 1, :]], axis=0)
        return lax.dot_general(lhs, rhs, dims,
                               precision=lax.Precision.HIGHEST,
                               preferred_element_type=jnp.float32)

    dx = diff(0)
    dy = diff(1)
    dz = diff(2)
    d2 = dx * dx + dy * dy + dz * dz
    lin = lax.broadcasted_iota(jnp.int32, (N, QK), 0)
    slot = lax.broadcasted_iota(jnp.int32, (NSAMPLE, QK), 0)
    inf = jnp.float32(jnp.inf)
    idxs0 = jnp.zeros((NSAMPLE, QK), jnp.int32)

    def body(k, st):
        d2, idxs = st
        m = jnp.min(d2, axis=0, keepdims=True)
        j = jnp.min(jnp.where(d2 == m, lin, N), axis=0, keepdims=True)
        idxs = jnp.where(slot == k, j, idxs)
        return (jnp.where(lin == j, inf, d2), idxs)

    _, idxs = lax.fori_loop(0, NSAMPLE, body, (d2, idxs0))
    idx_ref[0] = idxs
    gidx_ref[0] = idxs + b * N


def _knn(xyz, nxq, interpret=False):
    nq = NPOINT // QK
    return pl.pallas_call(
        _knn_body,
        grid=(B, nq),
        in_specs=[
            pl.BlockSpec((1, N, 3), lambda b, q: (b, 0, 0)),
            pl.BlockSpec((1, 3, QK), lambda b, q: (b, 0, q)),
        ],
        out_specs=(
            pl.BlockSpec((1, NSAMPLE, QK), lambda b, q: (b, 0, q)),
            pl.BlockSpec((1, NSAMPLE, QK), lambda b, q: (b, 0, q)),
        ),
        out_shape=(jax.ShapeDtypeStruct((B, NSAMPLE, NPOINT), jnp.int32),
                   jax.ShapeDtypeStruct((B, NSAMPLE, NPOINT), jnp.int32)),
        interpret=interpret,
    )(xyz, nxq)


_NW = 32
_RPW = TOT // _NW
_CH = 128
_NCH = _RPW // _CH


def _sc_gather(table, flat_idx):
    mesh = plsc.VectorSubcoreMesh(core_axis_name="c", subcore_axis_name="s")

    @functools.partial(
        pl.kernel,
        mesh=mesh,
        out_type=jax.ShapeDtypeStruct((TOT, CPAD), jnp.float32),
        compiler_params=pltpu.CompilerParams(use_tc_tiling_on_sc=True),
        scratch_types=[
            pltpu.VMEM((_CH,), jnp.int32),
            pltpu.VMEM((_CH, CPAD), jnp.float32),
            pltpu.SemaphoreType.DMA,
        ],
    )
    def k(table_hbm, idx_hbm, out_hbm, idx_v, rows_v, sem):
        wid = lax.axis_index("s") * 2 + lax.axis_index("c")

        def body(c, carry):
            base = wid * _RPW + c * _CH
            pltpu.sync_copy(idx_hbm.at[pl.ds(base, _CH)], idx_v)
            pltpu.async_copy(table_hbm.at[idx_v], rows_v, sem).wait()
            pltpu.sync_copy(rows_v, out_hbm.at[pl.ds(base, _CH)])
            return carry

        lax.fori_loop(0, _NCH, body, 0)

    return k(table, flat_idx)


def _mlp_body(feat_ref, nxq_ref, w0_ref, w1_ref, w2_ref,
              b0_ref, g0_ref, e0_ref, b1_ref, g1_ref, e1_ref,
              b2_ref, g2_ref, e2_ref, out_ref):
    rs = jnp.float32(1.0 / (1.0 + BN_EPS) ** 0.5)
    s0 = g0_ref[...] * rs
    s1 = g1_ref[...] * rs
    s2 = g2_ref[...] * rs
    w0 = w0_ref[...] * s0
    w1 = w1_ref[...] * s1
    w2 = w2_ref[...] * s2
    c0 = b0_ref[...] * s0 + e0_ref[...]
    c1 = b1_ref[...] * s1 + e1_ref[...]
    c2 = b2_ref[...] * s2 + e2_ref[...]

    q = nxq_ref[0]
    qc = lax.dot_general(q, w0[0:3, :], (((0,), (0,)), ((), ())),
                         preferred_element_type=jnp.float32)
    qce = jnp.broadcast_to(qc[:, None, :], (QB, NSAMPLE, 128))
    qce = qce.reshape(QB * NSAMPLE, 128)

    f = feat_ref[...]
    h = jnp.dot(f, w0, preferred_element_type=jnp.float32) + c0 - qce
    h = jnp.maximum(h, 0.0)
    h = jnp.dot(h, w1, preferred_element_type=jnp.float32) + c1
    h = jnp.maximum(h, 0.0)
    h = jnp.dot(h, w2, preferred_element_type=jnp.float32) + c2
    h = jnp.maximum(h, 0.0)
    pooled = jnp.max(h.reshape(QB, NSAMPLE, 256), axis=1)
    out_ref[0] = pooled


def _mlp(feats, nxq, w0p, w1, w2, b0, g0, e0, b1, g1, e1, b2, g2, e2,
         interpret=False):
    nq = NPOINT // QB
    full = lambda s: pl.BlockSpec(s, lambda b, q: tuple(0 for _ in s))
    return pl.pallas_call(
        _mlp_body,
        grid=(B, nq),
        in_specs=[
            pl.BlockSpec((QB * NSAMPLE, CPAD),
                         lambda b, q: (b * (NPOINT // QB) + q, 0)),
            pl.BlockSpec((1, 3, QB), lambda b, q: (b, 0, q)),
            full((CPAD, 128)), full((128, 128)), full((128, 256)),
            full((1, 128)), full((1, 128)), full((1, 128)),
            full((1, 128)), full((1, 128)), full((1, 128)),
            full((1, 256)), full((1, 256)), full((1, 256)),
        ],
        out_specs=pl.BlockSpec((1, QB, 256), lambda b, q: (b, q, 0)),
        out_shape=jax.ShapeDtypeStruct((B, NPOINT, 256), jnp.float32),
        interpret=interpret,
    )(feats, nxq, w0p, w1, w2, b0, g0, e0, b1, g1, e1, b2, g2, e2)


def kernel(xyz, points, training, W0, b0, gamma0, beta0,
           W1, b1, gamma1, beta1, W2, b2, gamma2, beta2):
    xyzt = jnp.transpose(xyz, (2, 0, 1))
    fps_idx, nxq3 = _fps(xyzt)
    nxq = jnp.transpose(nxq3, (1, 0, 2))
    idx_t, gidx_t = _knn(xyz, nxq)

    idx = jnp.transpose(idx_t, (0, 2, 1))
    flat_idx = jnp.transpose(gidx_t, (0, 2, 1)).reshape(TOT)

    table = jnp.concatenate(
        [xyz, points, jnp.zeros((B, N, CPAD - CIN), jnp.float32)],
        axis=-1).reshape(B * N, CPAD)
    feats = _sc_gather(table, flat_idx)

    w0p = jnp.concatenate([W0, jnp.zeros((CPAD - CIN, 128), jnp.float32)],
                          axis=0)
    new_points = _mlp(feats, nxq, w0p, W1, W2,
                      b0.reshape(1, -1), gamma0.reshape(1, -1),
                      beta0.reshape(1, -1),
                      b1.reshape(1, -1), gamma1.reshape(1, -1),
                      beta1.reshape(1, -1),
                      b2.reshape(1, -1), gamma2.reshape(1, -1),
                      beta2.reshape(1, -1))

    new_xyz = jnp.transpose(nxq3, (1, 2, 0))
    return (new_xyz, new_points, idx)


# speedup vs baseline: 1.4395x; 1.4395x over previous
import functools

import jax
import jax.numpy as jnp
from jax import lax
from jax.experimental import pallas as pl
from jax.experimental.pallas import tpu as pltpu
from jax.experimental.pallas import tpu_sc as plsc

B, N, CPT = 8, 4096, 64
NPOINT, NSAMPLE = 512, 32
BN_EPS = 1e-3
CIN = 3 + CPT
CPAD = 128
QB = 128
QK = 256
TOT = B * NPOINT * NSAMPLE


def _fps_body(xyzt_ref, idx_ref, nxq_ref):
    x = xyzt_ref[0]
    y = xyzt_ref[1]
    z = xyzt_ref[2]
    lane = lax.broadcasted_iota(jnp.int32, (B, N), 1)
    slot = lax.broadcasted_iota(jnp.int32, (B, NPOINT), 1)

    cx0 = x[:, 0:1]
    cy0 = y[:, 0:1]
    cz0 = z[:, 0:1]
    idxs0 = jnp.zeros((B, NPOINT), jnp.int32)
    qx0 = jnp.where(slot == 0, cx0, 0.0)
    qy0 = jnp.where(slot == 0, cy0, 0.0)
    qz0 = jnp.where(slot == 0, cz0, 0.0)
    dists0 = jnp.full((B, N), 1e10, jnp.float32)

    def body(i, st):
        dists, idxs, qx, qy, qz, cx, cy, cz = st
        dx = x - cx
        dy = y - cy
        dz = z - cz
        d = dx * dx + dy * dy + dz * dz
        dists = jnp.minimum(dists, d)
        m = jnp.max(dists, axis=1, keepdims=True)
        nxt = jnp.min(jnp.where(dists == m, lane, N), axis=1, keepdims=True)
        sel = lane == nxt
        cx = jnp.sum(jnp.where(sel, x, 0.0), axis=1, keepdims=True)
        cy = jnp.sum(jnp.where(sel, y, 0.0), axis=1, keepdims=True)
        cz = jnp.sum(jnp.where(sel, z, 0.0), axis=1, keepdims=True)
        put = slot == i
        idxs = jnp.where(put, nxt, idxs)
        qx = jnp.where(put, cx, qx)
        qy = jnp.where(put, cy, qy)
        qz = jnp.where(put, cz, qz)
        return (dists, idxs, qx, qy, qz, cx, cy, cz)

    st = lax.fori_loop(1, NPOINT, body,
                       (dists0, idxs0, qx0, qy0, qz0, cx0, cy0, cz0),
                       unroll=2)
    _, idxs, qx, qy, qz, _, _, _ = st
    idx_ref[...] = idxs
    nxq_ref[0] = qx
    nxq_ref[1] = qy
    nxq_ref[2] = qz


def _fps(xyzt, interpret=False):
    return pl.pallas_call(
        _fps_body,
        out_shape=(jax.ShapeDtypeStruct((B, NPOINT), jnp.int32),
                   jax.ShapeDtypeStruct((3, B, NPOINT), jnp.float32)),
        interpret=interpret,
    )(xyzt)


def _knn_body(xyz_ref, nxq_ref, idx_ref, gidx_ref):
    b = pl.program_id(0)
    xx = xyz_ref[0, :, 0:1]
    yy = xyz_ref[0, :, 1:2]
    zz = xyz_ref[0, :, 2:3]
    qx = nxq_ref[0, 0:1, :]
    qy = nxq_ref[0, 1:2, :]
    qz = nxq_ref[0, 2:3, :]
    dx = xx - qx
    dy = yy - qy
    dz = zz - qz
    d2 = dx * dx + dy * dy + dz * dz
    lin = lax.broadcasted_iota(jnp.int32, (N, QK), 0)
    slot = lax.broadcasted_iota(jnp.int32, (NSAMPLE, QK), 0)
    inf = jnp.float32(jnp.inf)
    idxs0 = jnp.zeros((NSAMPLE, QK), jnp.int32)

    def body(k, st):
        d2, idxs = st
        m = jnp.min(d2, axis=0, keepdims=True)
        j = jnp.min(jnp.where(d2 == m, lin, N), axis=0, keepdims=True)
        idxs = jnp.where(slot == k, j, idxs)
        return (jnp.where(lin == j, inf, d2), idxs)

    _, idxs = lax.fori_loop(0, NSAMPLE, body, (d2, idxs0), unroll=4)
    idx_ref[0] = idxs
    gidx_ref[0] = idxs + b * N


def _knn(xyz, nxq, interpret=False):
    nq = NPOINT // QK
    return pl.pallas_call(
        _knn_body,
        grid=(B, nq),
        in_specs=[
            pl.BlockSpec((1, N, 3), lambda b, q: (b, 0, 0)),
            pl.BlockSpec((1, 3, QK), lambda b, q: (b, 0, q)),
        ],
        out_specs=(
            pl.BlockSpec((1, NSAMPLE, QK), lambda b, q: (b, 0, q)),
            pl.BlockSpec((1, NSAMPLE, QK), lambda b, q: (b, 0, q)),
        ),
        out_shape=(jax.ShapeDtypeStruct((B, NSAMPLE, NPOINT), jnp.int32),
                   jax.ShapeDtypeStruct((B, NSAMPLE, NPOINT), jnp.int32)),
        interpret=interpret,
    )(xyz, nxq)


_NW = 32
_RPW = TOT // _NW
_CH = 128
_NCH = _RPW // _CH


def _sc_gather(table, flat_idx):
    mesh = plsc.VectorSubcoreMesh(core_axis_name="c", subcore_axis_name="s")

    @functools.partial(
        pl.kernel,
        mesh=mesh,
        out_type=jax.ShapeDtypeStruct((TOT, CPAD), jnp.float32),
        compiler_params=pltpu.CompilerParams(use_tc_tiling_on_sc=True),
        scratch_types=[
            pltpu.VMEM((_CH,), jnp.int32),
            pltpu.VMEM((_CH, CPAD), jnp.float32),
            pltpu.VMEM((_CH,), jnp.int32),
            pltpu.VMEM((_CH, CPAD), jnp.float32),
            pltpu.SemaphoreType.DMA,
            pltpu.SemaphoreType.DMA,
        ],
    )
    def k(table_hbm, idx_hbm, out_hbm, idx_v0, rows_v0, idx_v1, rows_v1,
          sem0, sem1):
        wid = lax.axis_index("s") * 2 + lax.axis_index("c")
        bufs = ((idx_v0, rows_v0, sem0), (idx_v1, rows_v1, sem1))
        base0 = wid * _RPW

        idx_v, rows_v, sem = bufs[0]
        pltpu.sync_copy(idx_hbm.at[pl.ds(base0, _CH)], idx_v)
        g = pltpu.async_copy(table_hbm.at[idx_v], rows_v, sem)
        for c in range(_NCH):
            if c + 1 < _NCH:
                idx_n, rows_n, sem_n = bufs[(c + 1) % 2]
                pltpu.sync_copy(
                    idx_hbm.at[pl.ds(base0 + (c + 1) * _CH, _CH)], idx_n)
                g_n = pltpu.async_copy(table_hbm.at[idx_n], rows_n, sem_n)
            g.wait()
            idx_v, rows_v, sem = bufs[c % 2]
            pltpu.sync_copy(rows_v, out_hbm.at[pl.ds(base0 + c * _CH, _CH)])
            if c + 1 < _NCH:
                g = g_n

    return k(table, flat_idx)


def _mlp_body(feat_ref, nxq_ref, w0_ref, w1_ref, w2_ref,
              b0_ref, g0_ref, e0_ref, b1_ref, g1_ref, e1_ref,
              b2_ref, g2_ref, e2_ref, out_ref):
    rs = jnp.float32(1.0 / (1.0 + BN_EPS) ** 0.5)
    s0 = g0_ref[...] * rs
    s1 = g1_ref[...] * rs
    s2 = g2_ref[...] * rs
    w0 = w0_ref[...] * s0
    w1 = w1_ref[...] * s1
    w2 = w2_ref[...] * s2
    c0 = b0_ref[...] * s0 + e0_ref[...]
    c1 = b1_ref[...] * s1 + e1_ref[...]
    c2 = b2_ref[...] * s2 + e2_ref[...]

    q = nxq_ref[0]
    qc = lax.dot_general(q, w0[0:3, :], (((0,), (0,)), ((), ())),
                         preferred_element_type=jnp.float32)
    qce = jnp.broadcast_to(qc[:, None, :], (QB, NSAMPLE, 128))
    qce = qce.reshape(QB * NSAMPLE, 128)

    f = feat_ref[...]
    h = jnp.dot(f, w0, preferred_element_type=jnp.float32) + c0 - qce
    h = jnp.maximum(h, 0.0)
    h = jnp.dot(h, w1, preferred_element_type=jnp.float32) + c1
    h = jnp.maximum(h, 0.0)
    h = jnp.dot(h, w2, preferred_element_type=jnp.float32) + c2
    h = jnp.maximum(h, 0.0)
    pooled = jnp.max(h.reshape(QB, NSAMPLE, 256), axis=1)
    out_ref[0] = pooled


def _mlp(feats, nxq, w0p, w1, w2, b0, g0, e0, b1, g1, e1, b2, g2, e2,
         interpret=False):
    nq = NPOINT // QB
    full = lambda s: pl.BlockSpec(s, lambda b, q: tuple(0 for _ in s))
    return pl.pallas_call(
        _mlp_body,
        grid=(B, nq),
        in_specs=[
            pl.BlockSpec((QB * NSAMPLE, CPAD),
                         lambda b, q: (b * (NPOINT // QB) + q, 0)),
            pl.BlockSpec((1, 3, QB), lambda b, q: (b, 0, q)),
            full((CPAD, 128)), full((128, 128)), full((128, 256)),
            full((1, 128)), full((1, 128)), full((1, 128)),
            full((1, 128)), full((1, 128)), full((1, 128)),
            full((1, 256)), full((1, 256)), full((1, 256)),
        ],
        out_specs=pl.BlockSpec((1, QB, 256), lambda b, q: (b, q, 0)),
        out_shape=jax.ShapeDtypeStruct((B, NPOINT, 256), jnp.float32),
        interpret=interpret,
    )(feats, nxq, w0p, w1, w2, b0, g0, e0, b1, g1, e1, b2, g2, e2)


def kernel(xyz, points, training, W0, b0, gamma0, beta0,
           W1, b1, gamma1, beta1, W2, b2, gamma2, beta2):
    xyzt = jnp.transpose(xyz, (2, 0, 1))
    fps_idx, nxq3 = _fps(xyzt)
    nxq = jnp.transpose(nxq3, (1, 0, 2))
    idx_t, gidx_t = _knn(xyz, nxq)

    idx = jnp.transpose(idx_t, (0, 2, 1))
    flat_idx = jnp.transpose(gidx_t, (0, 2, 1)).reshape(TOT)

    table = jnp.concatenate(
        [xyz, points, jnp.zeros((B, N, CPAD - CIN), jnp.float32)],
        axis=-1).reshape(B * N, CPAD)
    feats = _sc_gather(table, flat_idx)

    w0p = jnp.concatenate([W0, jnp.zeros((CPAD - CIN, 128), jnp.float32)],
                          axis=0)
    new_points = _mlp(feats, nxq, w0p, W1, W2,
                      b0.reshape(1, -1), gamma0.reshape(1, -1),
                      beta0.reshape(1, -1),
                      b1.reshape(1, -1), gamma1.reshape(1, -1),
                      beta1.reshape(1, -1),
                      b2.reshape(1, -1), gamma2.reshape(1, -1),
                      beta2.reshape(1, -1))

    new_xyz = jnp.transpose(nxq3, (1, 2, 0))
    return (new_xyz, new_points, idx)


# speedup vs baseline: 1.5222x; 1.0575x over previous
import functools

import jax
import jax.numpy as jnp
from jax import lax
from jax.experimental import pallas as pl
from jax.experimental.pallas import tpu as pltpu
from jax.experimental.pallas import tpu_sc as plsc

B, N, CPT = 8, 4096, 64
NPOINT, NSAMPLE = 512, 32
BN_EPS = 1e-3
CIN = 3 + CPT
CPAD = 128
QB = 128
QK = 256
TOT = B * NPOINT * NSAMPLE


def _fps_body(xyzt_ref, idx_ref, nxq_ref):
    x = xyzt_ref[0]
    y = xyzt_ref[1]
    z = xyzt_ref[2]
    lane = lax.broadcasted_iota(jnp.int32, (B, N), 1)
    slot = lax.broadcasted_iota(jnp.int32, (B, NPOINT), 1)

    cx0 = x[:, 0:1]
    cy0 = y[:, 0:1]
    cz0 = z[:, 0:1]
    idxs0 = jnp.zeros((B, NPOINT), jnp.int32)
    qx0 = jnp.where(slot == 0, cx0, 0.0)
    qy0 = jnp.where(slot == 0, cy0, 0.0)
    qz0 = jnp.where(slot == 0, cz0, 0.0)
    dists0 = jnp.full((B, N), 1e10, jnp.float32)

    def body(i, st):
        dists, idxs, qx, qy, qz, cx, cy, cz = st
        dx = x - cx
        dy = y - cy
        dz = z - cz
        d = dx * dx + dy * dy + dz * dz
        dists = jnp.minimum(dists, d)
        m = jnp.max(dists, axis=1, keepdims=True)
        nxt = jnp.min(jnp.where(dists == m, lane, N), axis=1, keepdims=True)
        sel = lane == nxt
        cx = jnp.sum(jnp.where(sel, x, 0.0), axis=1, keepdims=True)
        cy = jnp.sum(jnp.where(sel, y, 0.0), axis=1, keepdims=True)
        cz = jnp.sum(jnp.where(sel, z, 0.0), axis=1, keepdims=True)
        put = slot == i
        idxs = jnp.where(put, nxt, idxs)
        qx = jnp.where(put, cx, qx)
        qy = jnp.where(put, cy, qy)
        qz = jnp.where(put, cz, qz)
        return (dists, idxs, qx, qy, qz, cx, cy, cz)

    st = lax.fori_loop(1, NPOINT, body,
                       (dists0, idxs0, qx0, qy0, qz0, cx0, cy0, cz0),
                       unroll=4)
    _, idxs, qx, qy, qz, _, _, _ = st
    idx_ref[...] = idxs
    nxq_ref[0] = qx
    nxq_ref[1] = qy
    nxq_ref[2] = qz


def _fps(xyzt, interpret=False):
    return pl.pallas_call(
        _fps_body,
        out_shape=(jax.ShapeDtypeStruct((B, NPOINT), jnp.int32),
                   jax.ShapeDtypeStruct((3, B, NPOINT), jnp.float32)),
        interpret=interpret,
    )(xyzt)


def _knn_body(xyz_ref, nxq_ref, idx_ref, gidx_ref):
    b = pl.program_id(0)
    xx = xyz_ref[0, :, 0:1]
    yy = xyz_ref[0, :, 1:2]
    zz = xyz_ref[0, :, 2:3]
    qx = nxq_ref[0, 0:1, :]
    qy = nxq_ref[0, 1:2, :]
    qz = nxq_ref[0, 2:3, :]
    dx = xx - qx
    dy = yy - qy
    dz = zz - qz
    d2 = dx * dx + dy * dy + dz * dz
    lin = lax.broadcasted_iota(jnp.int32, (N, QK), 0)
    slot = lax.broadcasted_iota(jnp.int32, (NSAMPLE, QK), 0)
    inf = jnp.float32(jnp.inf)
    idxs0 = jnp.zeros((NSAMPLE, QK), jnp.int32)

    def body(k, st):
        d2, idxs = st
        m = jnp.min(d2, axis=0, keepdims=True)
        j = jnp.min(jnp.where(d2 == m, lin, N), axis=0, keepdims=True)
        idxs = jnp.where(slot == k, j, idxs)
        return (jnp.where(lin == j, inf, d2), idxs)

    _, idxs = lax.fori_loop(0, NSAMPLE, body, (d2, idxs0), unroll=8)
    idx_ref[0] = idxs
    gidx_ref[0] = idxs + b * N


def _knn(xyz, nxq, interpret=False):
    nq = NPOINT // QK
    return pl.pallas_call(
        _knn_body,
        grid=(B, nq),
        in_specs=[
            pl.BlockSpec((1, N, 3), lambda b, q: (b, 0, 0)),
            pl.BlockSpec((1, 3, QK), lambda b, q: (b, 0, q)),
        ],
        out_specs=(
            pl.BlockSpec((1, NSAMPLE, QK), lambda b, q: (b, 0, q)),
            pl.BlockSpec((1, NSAMPLE, QK), lambda b, q: (b, 0, q)),
        ),
        out_shape=(jax.ShapeDtypeStruct((B, NSAMPLE, NPOINT), jnp.int32),
                   jax.ShapeDtypeStruct((B, NSAMPLE, NPOINT), jnp.int32)),
        interpret=interpret,
    )(xyz, nxq)


_NW = 32
_RPW = TOT // _NW
_CH = 128
_NCH = _RPW // _CH


def _sc_gather(table, flat_idx):
    mesh = plsc.VectorSubcoreMesh(core_axis_name="c", subcore_axis_name="s")

    @functools.partial(
        pl.kernel,
        mesh=mesh,
        out_type=jax.ShapeDtypeStruct((TOT, CPAD), jnp.float32),
        compiler_params=pltpu.CompilerParams(use_tc_tiling_on_sc=True),
        scratch_types=[
            pltpu.VMEM((_CH,), jnp.int32),
            pltpu.VMEM((_CH, CPAD), jnp.float32),
            pltpu.VMEM((_CH,), jnp.int32),
            pltpu.VMEM((_CH, CPAD), jnp.float32),
            pltpu.SemaphoreType.DMA,
            pltpu.SemaphoreType.DMA,
        ],
    )
    def k(table_hbm, idx_hbm, out_hbm, idx_v0, rows_v0, idx_v1, rows_v1,
          sem0, sem1):
        wid = lax.axis_index("s") * 2 + lax.axis_index("c")
        bufs = ((idx_v0, rows_v0, sem0), (idx_v1, rows_v1, sem1))
        base0 = wid * _RPW

        idx_v, rows_v, sem = bufs[0]
        pltpu.sync_copy(idx_hbm.at[pl.ds(base0, _CH)], idx_v)
        g = pltpu.async_copy(table_hbm.at[idx_v], rows_v, sem)
        for c in range(_NCH):
            if c + 1 < _NCH:
                idx_n, rows_n, sem_n = bufs[(c + 1) % 2]
                pltpu.sync_copy(
                    idx_hbm.at[pl.ds(base0 + (c + 1) * _CH, _CH)], idx_n)
                g_n = pltpu.async_copy(table_hbm.at[idx_n], rows_n, sem_n)
            g.wait()
            idx_v, rows_v, sem = bufs[c % 2]
            pltpu.sync_copy(rows_v, out_hbm.at[pl.ds(base0 + c * _CH, _CH)])
            if c + 1 < _NCH:
                g = g_n

    return k(table, flat_idx)


def _mlp_body(feat_ref, nxq_ref, w0_ref, w1_ref, w2_ref,
              b0_ref, g0_ref, e0_ref, b1_ref, g1_ref, e1_ref,
              b2_ref, g2_ref, e2_ref, out_ref):
    rs = jnp.float32(1.0 / (1.0 + BN_EPS) ** 0.5)
    s0 = g0_ref[...] * rs
    s1 = g1_ref[...] * rs
    s2 = g2_ref[...] * rs
    w0 = w0_ref[...] * s0
    w1 = w1_ref[...] * s1
    w2 = w2_ref[...] * s2
    c0 = b0_ref[...] * s0 + e0_ref[...]
    c1 = b1_ref[...] * s1 + e1_ref[...]
    c2 = b2_ref[...] * s2 + e2_ref[...]

    q = nxq_ref[0]
    qc = lax.dot_general(q, w0[0:3, :], (((0,), (0,)), ((), ())),
                         preferred_element_type=jnp.float32)
    qce = jnp.broadcast_to(qc[:, None, :], (QB, NSAMPLE, 128))
    qce = qce.reshape(QB * NSAMPLE, 128)

    f = feat_ref[...]
    h = jnp.dot(f, w0, preferred_element_type=jnp.float32) + c0 - qce
    h = jnp.maximum(h, 0.0)
    h = jnp.dot(h, w1, preferred_element_type=jnp.float32) + c1
    h = jnp.maximum(h, 0.0)
    h = jnp.dot(h, w2, preferred_element_type=jnp.float32) + c2
    h = jnp.maximum(h, 0.0)
    pooled = jnp.max(h.reshape(QB, NSAMPLE, 256), axis=1)
    out_ref[0] = pooled


def _mlp(feats, nxq, w0p, w1, w2, b0, g0, e0, b1, g1, e1, b2, g2, e2,
         interpret=False):
    nq = NPOINT // QB
    full = lambda s: pl.BlockSpec(s, lambda b, q: tuple(0 for _ in s))
    return pl.pallas_call(
        _mlp_body,
        grid=(B, nq),
        in_specs=[
            pl.BlockSpec((QB * NSAMPLE, CPAD),
                         lambda b, q: (b * (NPOINT // QB) + q, 0)),
            pl.BlockSpec((1, 3, QB), lambda b, q: (b, 0, q)),
            full((CPAD, 128)), full((128, 128)), full((128, 256)),
            full((1, 128)), full((1, 128)), full((1, 128)),
            full((1, 128)), full((1, 128)), full((1, 128)),
            full((1, 256)), full((1, 256)), full((1, 256)),
        ],
        out_specs=pl.BlockSpec((1, QB, 256), lambda b, q: (b, q, 0)),
        out_shape=jax.ShapeDtypeStruct((B, NPOINT, 256), jnp.float32),
        interpret=interpret,
    )(feats, nxq, w0p, w1, w2, b0, g0, e0, b1, g1, e1, b2, g2, e2)


def kernel(xyz, points, training, W0, b0, gamma0, beta0,
           W1, b1, gamma1, beta1, W2, b2, gamma2, beta2):
    xyzt = jnp.transpose(xyz, (2, 0, 1))
    fps_idx, nxq3 = _fps(xyzt)
    nxq = jnp.transpose(nxq3, (1, 0, 2))
    idx_t, gidx_t = _knn(xyz, nxq)

    idx = jnp.transpose(idx_t, (0, 2, 1))
    flat_idx = jnp.transpose(gidx_t, (0, 2, 1)).reshape(TOT)

    table = jnp.concatenate(
        [xyz, points, jnp.zeros((B, N, CPAD - CIN), jnp.float32)],
        axis=-1).reshape(B * N, CPAD)
    feats = _sc_gather(table, flat_idx)

    w0p = jnp.concatenate([W0, jnp.zeros((CPAD - CIN, 128), jnp.float32)],
                          axis=0)
    new_points = _mlp(feats, nxq, w0p, W1, W2,
                      b0.reshape(1, -1), gamma0.reshape(1, -1),
                      beta0.reshape(1, -1),
                      b1.reshape(1, -1), gamma1.reshape(1, -1),
                      beta1.reshape(1, -1),
                      b2.reshape(1, -1), gamma2.reshape(1, -1),
                      beta2.reshape(1, -1))

    new_xyz = jnp.transpose(nxq3, (1, 2, 0))
    return (new_xyz, new_points, idx)
